# baseline (speedup 1.0000x reference)
import jax
import jax.numpy as jnp
from jax import lax
from jax.experimental import pallas as pl
from jax.experimental.pallas import tpu as pltpu
from jax.experimental.pallas import tpu_sc as plsc

D_HALF_K = 384
H_K = 32
W_K = 32
N_K = H_K * W_K
B_K = 32


def _sc_body(row_hbm, col_hbm, hw_hbm, out_hbm, hw_v, rows_v,
             sem_hw, sem_r, sem_c, sem_w):
    wid = lax.axis_index("s") * 2 + lax.axis_index("c")
    base = wid * B_K
    cp_hw = pltpu.async_copy(hw_hbm, hw_v, sem_hw)
    iota = lax.iota(jnp.int32, 16)
    widv = jnp.broadcast_to(wid, (16,)).astype(jnp.int32)
    g0 = pltpu.async_copy(row_hbm.at[widv], rows_v.at[pl.ds(0, 16)], sem_r)
    g1 = pltpu.async_copy(col_hbm.at[pl.ds(0, B_K)], rows_v.at[pl.ds(16, B_K)],
                          sem_c)
    g0.wait()
    w0 = pltpu.async_copy(rows_v.at[pl.ds(0, 16)],
                          out_hbm.at[pl.ds(base, 16), pl.ds(0, D_HALF_K)],
                          sem_w)
    w1 = pltpu.async_copy(rows_v.at[pl.ds(0, 16)],
                          out_hbm.at[pl.ds(base + 16, 16), pl.ds(0, D_HALF_K)],
                          sem_w)
    g1.wait()
    w2 = pltpu.async_copy(rows_v.at[pl.ds(16, B_K)],
                          out_hbm.at[pl.ds(base, B_K), pl.ds(D_HALF_K, D_HALF_K)],
                          sem_w)
    cp_hw.wait()
    hwv = hw_v[...]
    hm1 = hwv[0] - 1
    wm1 = hwv[1] - 1
    w0.wait()
    w1.wait()
    w2.wait()

    @pl.when(hm1 < B_K - 1)
    def _reclamp_rows():
        idx_r = jnp.minimum(widv, jnp.maximum(hm1, 0))
        pltpu.async_copy(row_hbm.at[idx_r], rows_v.at[pl.ds(0, 16)], sem_r).wait()
        pltpu.async_copy(rows_v.at[pl.ds(0, 16)],
                         out_hbm.at[pl.ds(base, 16), pl.ds(0, D_HALF_K)],
                         sem_w).wait()
        pltpu.async_copy(rows_v.at[pl.ds(0, 16)],
                         out_hbm.at[pl.ds(base + 16, 16), pl.ds(0, D_HALF_K)],
                         sem_w).wait()

    @pl.when(wm1 < B_K - 1)
    def _reclamp_cols():
        idx_c0 = jnp.minimum(iota, jnp.maximum(wm1, 0))
        idx_c1 = jnp.minimum(iota + 16, jnp.maximum(wm1, 0))
        pltpu.async_copy(col_hbm.at[idx_c0], rows_v.at[pl.ds(16, 16)], sem_c).wait()
        pltpu.async_copy(col_hbm.at[idx_c1], rows_v.at[pl.ds(32, 16)], sem_c).wait()
        pltpu.async_copy(rows_v.at[pl.ds(16, B_K)],
                         out_hbm.at[pl.ds(base, B_K), pl.ds(D_HALF_K, D_HALF_K)],
                         sem_w).wait()


def kernel(h, w, row_table, col_table):
    hw8 = jnp.zeros((16,), jnp.int32).at[0].set(h).at[1].set(w)
    k = pl.kernel(
        _sc_body,
        mesh=plsc.VectorSubcoreMesh(core_axis_name="c", subcore_axis_name="s"),
        out_type=jax.ShapeDtypeStruct((N_K, 2 * D_HALF_K), jnp.float32),
        scratch_types=[
            pltpu.VMEM((16,), jnp.int32),
            pltpu.VMEM((16 + B_K, D_HALF_K), jnp.float32),
            pltpu.SemaphoreType.DMA,
            pltpu.SemaphoreType.DMA,
            pltpu.SemaphoreType.DMA,
            pltpu.SemaphoreType.DMA,
        ],
    )
    return k(row_table, col_table, hw8)


# speedup vs baseline: 1.0152x; 1.0152x over previous
import jax
import jax.numpy as jnp
from jax import lax
from jax.experimental import pallas as pl
from jax.experimental.pallas import tpu as pltpu
from jax.experimental.pallas import tpu_sc as plsc

D_HALF_K = 384
H_K = 32
W_K = 32
N_K = H_K * W_K
B_K = 32


def _sc_body(row_hbm, col_hbm, hw_hbm, out_hbm, hw_v, buf_v,
             sem_hw, sem_g, sem_r, sem_c):
    wid = lax.axis_index("s") * 2 + lax.axis_index("c")
    cp_hw = pltpu.async_copy(hw_hbm, hw_v, sem_hw)
    iota = lax.iota(jnp.int32, 16)
    gr = pltpu.async_copy(row_hbm.at[pl.ds(0, B_K)], buf_v.at[pl.ds(0, B_K)],
                          sem_r)
    gc = pltpu.async_copy(col_hbm.at[pl.ds(0, B_K)], buf_v.at[pl.ds(B_K, B_K)],
                          sem_c)
    gr.wait()
    wr = pltpu.async_copy(
        buf_v.at[pl.ds(0, B_K)],
        out_hbm.at[pl.ds(0, H_K), wid, pl.ds(0, D_HALF_K)], sem_r)
    gc.wait()
    wc = pltpu.async_copy(
        buf_v.at[pl.ds(B_K, B_K)],
        out_hbm.at[wid, pl.ds(0, W_K), pl.ds(D_HALF_K, D_HALF_K)], sem_c)
    cp_hw.wait()
    hwv = hw_v[...]
    hm1 = hwv[0] - 1
    wm1 = hwv[1] - 1
    wr.wait()
    wc.wait()

    @pl.when(hm1 < H_K - 1)
    def _reclamp_rows():
        idx0 = jnp.minimum(iota, jnp.maximum(hm1, 0))
        idx1 = jnp.minimum(iota + 16, jnp.maximum(hm1, 0))
        pltpu.async_copy(row_hbm.at[idx0], buf_v.at[pl.ds(0, 16)], sem_g).wait()
        pltpu.async_copy(row_hbm.at[idx1], buf_v.at[pl.ds(16, 16)], sem_g).wait()
        pltpu.async_copy(
            buf_v.at[pl.ds(0, B_K)],
            out_hbm.at[pl.ds(0, H_K), wid, pl.ds(0, D_HALF_K)], sem_r).wait()

    @pl.when(wm1 < W_K - 1)
    def _reclamp_cols():
        idx0 = jnp.minimum(iota, jnp.maximum(wm1, 0))
        idx1 = jnp.minimum(iota + 16, jnp.maximum(wm1, 0))
        pltpu.async_copy(col_hbm.at[idx0], buf_v.at[pl.ds(B_K, 16)], sem_g).wait()
        pltpu.async_copy(col_hbm.at[idx1], buf_v.at[pl.ds(B_K + 16, 16)],
                         sem_g).wait()
        pltpu.async_copy(
            buf_v.at[pl.ds(B_K, B_K)],
            out_hbm.at[wid, pl.ds(0, W_K), pl.ds(D_HALF_K, D_HALF_K)],
            sem_c).wait()


def kernel(h, w, row_table, col_table):
    hw8 = jnp.zeros((16,), jnp.int32).at[0].set(h).at[1].set(w)
    k = pl.kernel(
        _sc_body,
        mesh=plsc.VectorSubcoreMesh(core_axis_name="c", subcore_axis_name="s"),
        out_type=jax.ShapeDtypeStruct((H_K, W_K, 2 * D_HALF_K), jnp.float32),
        scratch_types=[
            pltpu.VMEM((16,), jnp.int32),
            pltpu.VMEM((2 * B_K, D_HALF_K), jnp.float32),
            pltpu.SemaphoreType.DMA,
            pltpu.SemaphoreType.DMA,
            pltpu.SemaphoreType.DMA,
            pltpu.SemaphoreType.DMA,
        ],
    )
    return k(row_table, col_table, hw8).reshape(N_K, 2 * D_HALF_K)
